# Initial kernel scaffold; baseline (speedup 1.0000x reference)
import jax
import jax.numpy as jnp
from jax.experimental import pallas as pl


def kernel(x, edge_index, u, batch, W1, b1, W2, b2):
    raise NotImplementedError("write your pallas kernel here")


# speedup vs baseline: 2.1262x; 2.1262x over previous
import dataclasses

import jax
import jax.numpy as jnp
from jax import lax
from jax.experimental import pallas as pl
from jax.experimental.pallas import tpu as pltpu
from jax.experimental.pallas import tpu_sc as plsc

N = 10000
F = 256
B = 256
U = 128
HIDDEN = 2048
OUT = 512

NC = 2
NS = 16
NW = NC * NS
L = 16
NP = 10240
NT = NP // L
TPW = NT // NW
FV = F // L
ASZ = (B + 1) * F


def _sc_segment_sums(xr, batch_pad):
    mesh = plsc.VectorSubcoreMesh(core_axis_name="c", subcore_axis_name="s")
    cp = pltpu.CompilerParams()
    if "needs_layout_passes" in pltpu.CompilerParams.__dataclass_fields__:
        cp = dataclasses.replace(cp, needs_layout_passes=False)

    @pl.kernel(
        compiler_params=cp,
        out_type=jax.ShapeDtypeStruct((NW, ASZ), jnp.float32),
        mesh=mesh,
        scratch_types=[
            pltpu.VMEM((TPW * L,), jnp.int32),
            pltpu.VMEM((L * F,), jnp.float32),
            pltpu.VMEM((L * F,), jnp.float32),
            pltpu.VMEM((ASZ,), jnp.float32),
            pltpu.SemaphoreType.DMA,
            pltpu.SemaphoreType.DMA,
            pltpu.SemaphoreType.DMA,
        ],
    )
    def seg(xr_hbm, bat_hbm, sum_hbm, idx_v, tile0_v, tile1_v, acc_v,
            sem0, sem1, semi):
        c = lax.axis_index("c")
        s = lax.axis_index("s")
        w = c * NS + s
        t0 = w * TPW
        bufs = (tile0_v, tile1_v)
        sems = (sem0, sem1)

        idx_cp = pltpu.async_copy(bat_hbm.at[pl.ds(t0 * L, TPW * L)], idx_v,
                                  semi)
        cps = [pltpu.async_copy(xr_hbm.at[t0], tile0_v, sem0)]

        zeros16 = jnp.zeros((L,), jnp.float32)
        iota16 = lax.iota(jnp.int32, L)

        @pl.loop(0, ASZ, step=8 * L)
        def _(i):
            for d in range(8):
                acc_v[pl.ds(i + d * L, L)] = zeros16

        idx_cp.wait()
        for j in range(TPW):
            if j + 1 < TPW:
                cps.append(pltpu.async_copy(xr_hbm.at[t0 + j + 1],
                                            bufs[(j + 1) % 2],
                                            sems[(j + 1) % 2]))
            cps[j].wait()
            buf = bufs[j % 2]
            bases = idx_v[pl.ds(j * L, L)] * F

            @pl.loop(0, L)
            def _(r):
                base = lax.gather(
                    bases, jnp.full((L, 1), r, jnp.int32),
                    lax.GatherDimensionNumbers(
                        offset_dims=(), collapsed_slice_dims=(0,),
                        start_index_map=(0,)),
                    (1,), mode=lax.GatherScatterMode.PROMISE_IN_BOUNDS)
                addr0 = base + iota16
                for t in range(FV):
                    v = buf[pl.ds(r * F + t * L, L)]
                    plsc.addupdate_scatter(acc_v, [addr0 + t * L], v)

        pltpu.sync_copy(acc_v, sum_hbm.at[w])

    return seg(xr, batch_pad)


def _mlp_body(ps_ref, bat_ref, u_ref, w1u_ref, w1x_ref, b1_ref, w2_ref,
              b2_ref, o_ref):
    sums = jnp.sum(ps_ref[...], axis=0)[:B]
    segs = lax.broadcasted_iota(jnp.int32, (B, 1), 0).astype(jnp.float32)
    cnt = jnp.sum(jnp.where(bat_ref[...] == segs, 1.0, 0.0), axis=1,
                  keepdims=True)
    mean = sums / jnp.maximum(cnt, 1.0)
    h = jnp.dot(u_ref[...], w1u_ref[...], preferred_element_type=jnp.float32)
    h = h + jnp.dot(mean, w1x_ref[...], preferred_element_type=jnp.float32)
    h = jnp.maximum(h + b1_ref[...], 0.0)
    o_ref[...] = (jnp.dot(h, w2_ref[...], preferred_element_type=jnp.float32)
                  + b2_ref[...])


def _tc_mlp(part_sum, bat_row, u, w1u, w1x, b1, w2t, b2):
    return pl.pallas_call(
        _mlp_body,
        out_shape=jax.ShapeDtypeStruct((B, OUT), jnp.float32),
    )(part_sum, bat_row, u, w1u, w1x, b1, w2t, b2)


def kernel(x, edge_index, u, batch, W1, b1, W2, b2):
    del edge_index
    x = x.astype(jnp.float32)
    x_pad = jnp.concatenate([x, jnp.zeros((NP - N, F), jnp.float32)], axis=0)
    xr = x_pad.reshape(NT, L * F)
    bat_pad = jnp.concatenate(
        [batch.astype(jnp.int32), jnp.full((NP - N,), B, jnp.int32)])
    part_sum = _sc_segment_sums(xr, bat_pad)
    part_sum = part_sum.reshape(NW, B + 1, F)
    bat_row = bat_pad.astype(jnp.float32).reshape(1, NP)
    w1u = W1[:, :U].T
    w1x = W1[:, U:].T
    w2t = W2.T
    return _tc_mlp(part_sum, bat_row, u, w1u, w1x,
                   b1.reshape(1, HIDDEN), w2t, b2.reshape(1, OUT))


# speedup vs baseline: 2.3493x; 1.1049x over previous
import dataclasses

import jax
import jax.numpy as jnp
from jax import lax
from jax.experimental import pallas as pl
from jax.experimental.pallas import tpu as pltpu
from jax.experimental.pallas import tpu_sc as plsc

N = 10000
F = 256
B = 256
U = 128
HIDDEN = 2048
OUT = 512

NC = 2
NS = 16
NW = NC * NS
L = 16
NP = 10240
NT = NP // L
NTR = N // L
TPW = NT // NW
FV = F // L
ASZ = (B + 1) * F


def _sc_segment_sums(xr, batch_pad):
    mesh = plsc.VectorSubcoreMesh(core_axis_name="c", subcore_axis_name="s")
    cp = pltpu.CompilerParams()
    if "needs_layout_passes" in pltpu.CompilerParams.__dataclass_fields__:
        cp = dataclasses.replace(cp, needs_layout_passes=False)

    @pl.kernel(
        compiler_params=cp,
        out_type=jax.ShapeDtypeStruct((NW, ASZ), jnp.float32),
        mesh=mesh,
        scratch_types=[
            pltpu.VMEM((TPW * L,), jnp.int32),
            pltpu.VMEM((L * F,), jnp.float32),
            pltpu.VMEM((L * F,), jnp.float32),
            pltpu.VMEM((ASZ,), jnp.float32),
            pltpu.SemaphoreType.DMA,
            pltpu.SemaphoreType.DMA,
            pltpu.SemaphoreType.DMA,
        ],
    )
    def seg(xr_hbm, bat_hbm, sum_hbm, idx_v, tile0_v, tile1_v, acc_v,
            sem0, sem1, semi):
        c = lax.axis_index("c")
        s = lax.axis_index("s")
        w = c * NS + s
        t0 = w * TPW
        bufs = (tile0_v, tile1_v)
        sems = (sem0, sem1)

        idx_cp = pltpu.async_copy(bat_hbm.at[pl.ds(t0 * L, TPW * L)], idx_v,
                                  semi)
        cps = [pltpu.async_copy(xr_hbm.at[jnp.minimum(t0, NTR - 1)],
                                tile0_v, sem0)]

        zeros16 = jnp.zeros((L,), jnp.float32)
        iota16 = lax.iota(jnp.int32, L)

        @pl.loop(0, ASZ, step=8 * L)
        def _(i):
            for d in range(8):
                acc_v[pl.ds(i + d * L, L)] = zeros16

        idx_cp.wait()
        for j in range(TPW):
            if j + 1 < TPW:
                cps.append(pltpu.async_copy(
                    xr_hbm.at[jnp.minimum(t0 + j + 1, NTR - 1)],
                    bufs[(j + 1) % 2], sems[(j + 1) % 2]))
            cps[j].wait()
            buf = bufs[j % 2]
            bases = idx_v[pl.ds(j * L, L)] * F

            @pl.loop(0, L)
            def _(r):
                base = lax.gather(
                    bases, jnp.full((L, 1), r, jnp.int32),
                    lax.GatherDimensionNumbers(
                        offset_dims=(), collapsed_slice_dims=(0,),
                        start_index_map=(0,)),
                    (1,), mode=lax.GatherScatterMode.PROMISE_IN_BOUNDS)
                addr0 = base + iota16
                for t in range(FV):
                    v = buf[pl.ds(r * F + t * L, L)]
                    plsc.addupdate_scatter(acc_v, [addr0 + t * L], v)

        pltpu.sync_copy(acc_v, sum_hbm.at[w])

    return seg(xr, batch_pad)


def _nt_dot(a, b):
    return lax.dot_general(a, b, (((1,), (1,)), ((), ())),
                           preferred_element_type=jnp.float32)


def _mlp_body(ps_ref, bat_ref, u_ref, w1_ref, b1_ref, w2_ref, b2_ref, o_ref):
    sums = jnp.sum(ps_ref[...], axis=0)[:B]
    segs = lax.broadcasted_iota(jnp.int32, (B, 1), 0).astype(jnp.float32)
    cnt = jnp.sum(jnp.where(bat_ref[...] == segs, 1.0, 0.0), axis=1,
                  keepdims=True)
    mean = sums / jnp.maximum(cnt, 1.0)
    h = _nt_dot(u_ref[...], w1_ref[:, :U]) + _nt_dot(mean, w1_ref[:, U:])
    h = jnp.maximum(h + b1_ref[...], 0.0)
    o_ref[...] = _nt_dot(h, w2_ref[...]) + b2_ref[...]


def _tc_mlp(part_sum, bat_row, u, w1, b1, w2, b2):
    return pl.pallas_call(
        _mlp_body,
        out_shape=jax.ShapeDtypeStruct((B, OUT), jnp.float32),
    )(part_sum, bat_row, u, w1, b1, w2, b2)


def kernel(x, edge_index, u, batch, W1, b1, W2, b2):
    del edge_index
    x = x.astype(jnp.float32)
    xr = x.reshape(NTR, L * F)
    bat_pad = jnp.concatenate(
        [batch.astype(jnp.int32), jnp.full((NP - N,), B, jnp.int32)])
    part_sum = _sc_segment_sums(xr, bat_pad)
    part_sum = part_sum.reshape(NW, B + 1, F)
    bat_row = bat_pad.astype(jnp.float32).reshape(1, NP)
    return _tc_mlp(part_sum, bat_row, u, W1,
                   b1.reshape(1, HIDDEN), W2, b2.reshape(1, OUT))


# speedup vs baseline: 3.9665x; 1.6884x over previous
import dataclasses

import jax
import jax.numpy as jnp
from jax import lax
from jax.experimental import pallas as pl
from jax.experimental.pallas import tpu as pltpu
from jax.experimental.pallas import tpu_sc as plsc

N = 10000
F = 256
B = 256
U = 128
HIDDEN = 2048
OUT = 512

NC = 2
NS = 16
NW = NC * NS
L = 16
NP = 10240
NT = NP // L
NTR = N // L
TPW = NT // NW
FV = F // L
HL = F // 2
BPR = 272


def _sc_segment_sums(x, batch_pad):
    mesh = plsc.VectorSubcoreMesh(core_axis_name="c", subcore_axis_name="s")
    cp = pltpu.CompilerParams()
    if "needs_layout_passes" in pltpu.CompilerParams.__dataclass_fields__:
        cp = dataclasses.replace(cp, needs_layout_passes=False)

    @pl.kernel(
        compiler_params=cp,
        out_type=[
            jax.ShapeDtypeStruct((NW, BPR, HL), jnp.float32),
            jax.ShapeDtypeStruct((NW, BPR, HL), jnp.float32),
        ],
        mesh=mesh,
        scratch_types=[
            pltpu.VMEM((TPW * L,), jnp.int32),
            pltpu.VMEM((L, F), jnp.float32),
            pltpu.VMEM((L, F), jnp.float32),
            pltpu.VMEM((L, F), jnp.float32),
            pltpu.VMEM((L, F), jnp.float32),
            pltpu.VMEM((BPR, HL), jnp.float32),
            pltpu.VMEM((BPR, HL), jnp.float32),
            pltpu.SemaphoreType.DMA,
            pltpu.SemaphoreType.DMA,
            pltpu.SemaphoreType.DMA,
            pltpu.SemaphoreType.DMA,
            pltpu.SemaphoreType.DMA,
        ],
    )
    def seg(x_hbm, bat_hbm, lo_hbm, hi_hbm, idx_v, tile0_v, tile1_v, tile2_v,
            tile3_v, acc_lo, acc_hi, sem0, sem1, sem2, sem3, semi):
        c = lax.axis_index("c")
        s = lax.axis_index("s")
        w = c * NS + s
        t0 = w * TPW
        bufs = (tile0_v, tile1_v, tile2_v, tile3_v)
        sems = (sem0, sem1, sem2, sem3)
        NB = len(bufs)

        idx_cp = pltpu.async_copy(bat_hbm.at[pl.ds(t0 * L, TPW * L)], idx_v,
                                  semi)
        cps = [pltpu.async_copy(
            x_hbm.at[pl.ds(jnp.minimum(t0 + jj, NTR - 1) * L, L)],
            bufs[jj], sems[jj]) for jj in range(NB)]

        zeros16 = jnp.zeros((L,), jnp.float32)
        iota16 = lax.iota(jnp.int32, L)
        offs = [iota16 + d * L for d in range(FV // 2)]

        @pl.loop(0, BPR)
        def _(i):
            for d in range(FV // 2):
                acc_lo[i, pl.ds(d * L, L)] = zeros16
                acc_hi[i, pl.ds(d * L, L)] = zeros16

        idx_cp.wait()
        for j in range(TPW):
            if j + NB < TPW:
                cps.append(pltpu.async_copy(
                    x_hbm.at[pl.ds(jnp.minimum(t0 + j + NB, NTR - 1) * L, L)],
                    bufs[(j + NB) % NB], sems[(j + NB) % NB]))
            cps[j].wait()
            buf = bufs[j % NB]
            ids = idx_v[pl.ds(j * L, L)]

            @pl.loop(0, L)
            def _(r):
                row = lax.gather(
                    ids, jnp.full((L, 1), r, jnp.int32),
                    lax.GatherDimensionNumbers(
                        offset_dims=(), collapsed_slice_dims=(0,),
                        start_index_map=(0,)),
                    (1,), mode=lax.GatherScatterMode.PROMISE_IN_BOUNDS)
                vals = [buf[r, pl.ds(t * L, L)] for t in range(FV)]
                for t in range(FV):
                    acc = acc_lo if t < FV // 2 else acc_hi
                    plsc.addupdate_scatter(acc, [row, offs[t % (FV // 2)]],
                                           vals[t])

        pltpu.sync_copy(acc_lo, lo_hbm.at[w])
        pltpu.sync_copy(acc_hi, hi_hbm.at[w])

    return seg(x, batch_pad)


def _nt_dot(a, b):
    return lax.dot_general(a, b, (((1,), (1,)), ((), ())),
                           preferred_element_type=jnp.float32)


def _mlp_body(lo_ref, hi_ref, bat_ref, u_ref, w1_ref, b1_ref, w2_ref,
              b2_ref, o_ref):
    sums_lo = jnp.sum(lo_ref[...], axis=0)[:B]
    sums_hi = jnp.sum(hi_ref[...], axis=0)[:B]
    segs = lax.broadcasted_iota(jnp.int32, (B, 1), 0).astype(jnp.float32)
    cnt = jnp.sum(jnp.where(bat_ref[...] == segs, 1.0, 0.0), axis=1,
                  keepdims=True)
    inv = 1.0 / jnp.maximum(cnt, 1.0)
    h = (_nt_dot(u_ref[...], w1_ref[:, :U])
         + _nt_dot(sums_lo * inv, w1_ref[:, U:U + HL])
         + _nt_dot(sums_hi * inv, w1_ref[:, U + HL:]))
    h = jnp.maximum(h + b1_ref[...], 0.0)
    o_ref[...] = _nt_dot(h, w2_ref[...]) + b2_ref[...]


def _tc_mlp(part_lo, part_hi, bat_row, u, w1, b1, w2, b2):
    return pl.pallas_call(
        _mlp_body,
        out_shape=jax.ShapeDtypeStruct((B, OUT), jnp.float32),
    )(part_lo, part_hi, bat_row, u, w1, b1, w2, b2)


def kernel(x, edge_index, u, batch, W1, b1, W2, b2):
    del edge_index
    x = x.astype(jnp.float32)
    bat_pad = jnp.concatenate(
        [batch.astype(jnp.int32), jnp.full((NP - N,), B, jnp.int32)])
    part_lo, part_hi = _sc_segment_sums(x, bat_pad)
    bat_row = bat_pad.astype(jnp.float32).reshape(1, NP)
    return _tc_mlp(part_lo, part_hi, bat_row, u, W1,
                   b1.reshape(1, HIDDEN), W2, b2.reshape(1, OUT))


# speedup vs baseline: 4.0135x; 1.0118x over previous
import dataclasses

import jax
import jax.numpy as jnp
from jax import lax
from jax.experimental import pallas as pl
from jax.experimental.pallas import tpu as pltpu
from jax.experimental.pallas import tpu_sc as plsc

N = 10000
F = 256
B = 256
U = 128
HIDDEN = 2048
OUT = 512

NC = 2
NS = 16
NW = NC * NS
L = 16
NP = 10240
NT = NP // L
NTR = N // L
TPW = NT // NW
FV = F // L
HL = F // 2
BPR = 272


def _sc_segment_sums(x, batch_pad):
    mesh = plsc.VectorSubcoreMesh(core_axis_name="c", subcore_axis_name="s")
    cp = pltpu.CompilerParams()
    if "needs_layout_passes" in pltpu.CompilerParams.__dataclass_fields__:
        cp = dataclasses.replace(cp, needs_layout_passes=False)

    @pl.kernel(
        compiler_params=cp,
        out_type=[
            jax.ShapeDtypeStruct((NW, BPR, HL), jnp.float32),
            jax.ShapeDtypeStruct((NW, BPR, HL), jnp.float32),
        ],
        mesh=mesh,
        scratch_types=[
            pltpu.VMEM((TPW * L,), jnp.int32),
            pltpu.VMEM((L, F), jnp.float32),
            pltpu.VMEM((L, F), jnp.float32),
            pltpu.VMEM((L, F), jnp.float32),
            pltpu.VMEM((L, F), jnp.float32),
            pltpu.VMEM((BPR, HL), jnp.float32),
            pltpu.VMEM((BPR, HL), jnp.float32),
            pltpu.SemaphoreType.DMA,
            pltpu.SemaphoreType.DMA,
            pltpu.SemaphoreType.DMA,
            pltpu.SemaphoreType.DMA,
            pltpu.SemaphoreType.DMA,
        ],
    )
    def seg(x_hbm, bat_hbm, lo_hbm, hi_hbm, idx_v, tile0_v, tile1_v, tile2_v,
            tile3_v, acc_lo, acc_hi, sem0, sem1, sem2, sem3, semi):
        c = lax.axis_index("c")
        s = lax.axis_index("s")
        w = c * NS + s
        t0 = w * TPW
        bufs = (tile0_v, tile1_v, tile2_v, tile3_v)
        sems = (sem0, sem1, sem2, sem3)
        NB = len(bufs)

        idx_cp = pltpu.async_copy(bat_hbm.at[pl.ds(t0 * L, TPW * L)], idx_v,
                                  semi)
        cps = [pltpu.async_copy(
            x_hbm.at[pl.ds(jnp.minimum(t0 + jj, NTR - 1) * L, L)],
            bufs[jj], sems[jj]) for jj in range(NB)]

        zeros16 = jnp.zeros((L,), jnp.float32)
        iota16 = lax.iota(jnp.int32, L)
        offs = [iota16 + d * L for d in range(FV // 2)]

        @pl.loop(0, BPR)
        def _(i):
            for d in range(FV // 2):
                acc_lo[i, pl.ds(d * L, L)] = zeros16
                acc_hi[i, pl.ds(d * L, L)] = zeros16

        idx_cp.wait()
        for j in range(TPW):
            if j + NB < TPW:
                cps.append(pltpu.async_copy(
                    x_hbm.at[pl.ds(jnp.minimum(t0 + j + NB, NTR - 1) * L, L)],
                    bufs[(j + NB) % NB], sems[(j + NB) % NB]))
            cps[j].wait()
            buf = bufs[j % NB]
            ids = idx_v[pl.ds(j * L, L)]

            @pl.loop(0, L)
            def _(r):
                row = lax.gather(
                    ids, jnp.full((L, 1), r, jnp.int32),
                    lax.GatherDimensionNumbers(
                        offset_dims=(), collapsed_slice_dims=(0,),
                        start_index_map=(0,)),
                    (1,), mode=lax.GatherScatterMode.PROMISE_IN_BOUNDS)
                vals = [buf[r, pl.ds(t * L, L)] for t in range(FV)]
                for t in range(FV):
                    acc = acc_lo if t < FV // 2 else acc_hi
                    plsc.addupdate_scatter(acc, [row, offs[t % (FV // 2)]],
                                           vals[t])

        pltpu.sync_copy(acc_lo, lo_hbm.at[w])
        pltpu.sync_copy(acc_hi, hi_hbm.at[w])

    return seg(x, batch_pad)


def _nt_dot(a, b):
    return lax.dot_general(a.astype(jnp.bfloat16), b.astype(jnp.bfloat16),
                           (((1,), (1,)), ((), ())),
                           preferred_element_type=jnp.float32)


def _pre_body(bat_ref, u_ref, w1_ref, b1_ref, pre_ref, inv_ref):
    segs = lax.broadcasted_iota(jnp.int32, (B, 1), 0).astype(jnp.float32)
    cnt = jnp.sum(jnp.where(bat_ref[...] == segs, 1.0, 0.0), axis=1,
                  keepdims=True)
    inv_ref[...] = 1.0 / jnp.maximum(cnt, 1.0)
    pre_ref[...] = _nt_dot(u_ref[...], w1_ref[:, :U]) + b1_ref[...]


def _mlp_body(lo_ref, hi_ref, inv_ref, pre_ref, w1_ref, w2_ref, b2_ref,
              o_ref):
    sums_lo = jnp.sum(lo_ref[...], axis=0)[:B]
    sums_hi = jnp.sum(hi_ref[...], axis=0)[:B]
    inv = inv_ref[...]
    h = (pre_ref[...]
         + _nt_dot(sums_lo * inv, w1_ref[:, U:U + HL])
         + _nt_dot(sums_hi * inv, w1_ref[:, U + HL:]))
    h = jnp.maximum(h, 0.0)
    o_ref[...] = _nt_dot(h, w2_ref[...]) + b2_ref[...]


def kernel(x, edge_index, u, batch, W1, b1, W2, b2):
    del edge_index
    x = x.astype(jnp.float32)
    bat_pad = jnp.concatenate(
        [batch.astype(jnp.int32), jnp.full((NP - N,), B, jnp.int32)])
    part_lo, part_hi = _sc_segment_sums(x, bat_pad)
    bat_row = bat_pad.astype(jnp.float32).reshape(1, NP)
    pre, inv = pl.pallas_call(
        _pre_body,
        out_shape=[jax.ShapeDtypeStruct((B, HIDDEN), jnp.float32),
                   jax.ShapeDtypeStruct((B, 1), jnp.float32)],
    )(bat_row, u, W1, b1.reshape(1, HIDDEN))
    return pl.pallas_call(
        _mlp_body,
        out_shape=jax.ShapeDtypeStruct((B, OUT), jnp.float32),
    )(part_lo, part_hi, inv, pre, W1, W2, b2.reshape(1, OUT))
